# Initial kernel scaffold; baseline (speedup 1.0000x reference)
import functools

import jax
import jax.numpy as jnp
from jax import lax
from jax.experimental import pallas as pl
from jax.experimental.pallas import tpu as pltpu
from jax.experimental.pallas import tpu_sc as plsc

BATCH = 16384
N_FIELDS = 26
DIM = 32
TOTAL = BATCH * N_FIELDS
IDXW = 128
ROWS = TOTAL // IDXW
NC = 2
NS = 16
NW = NC * NS
ROWS_PER_W = ROWS // NW
KG = 13
NCHUNK = ROWS_PER_W // KG
CHUNK = KG * IDXW


def _body(idx_hbm, table_hbm, out_hbm, idx_v, rows_v, sem):
    wid = lax.axis_index("s") * NC + lax.axis_index("c")
    row0 = wid * ROWS_PER_W

    def chunk(i, _):
        r = row0 + i * KG
        pltpu.sync_copy(idx_hbm.at[pl.ds(r, KG)], idx_v)
        copies = []
        for j in range(KG):
            copies.append(
                pltpu.async_copy(
                    table_hbm.at[idx_v.at[j]],
                    rows_v.at[pl.ds(j * IDXW, IDXW)],
                    sem,
                )
            )
        for c in copies:
            c.wait()
        pltpu.sync_copy(rows_v, out_hbm.at[pl.ds(r * IDXW, CHUNK)])
        return 0

    lax.fori_loop(0, NCHUNK, chunk, 0)


_emb = functools.partial(
    pl.kernel,
    out_type=jax.ShapeDtypeStruct((TOTAL, DIM), jnp.float32),
    mesh=plsc.VectorSubcoreMesh(core_axis_name="c", subcore_axis_name="s"),
    scratch_types=[
        pltpu.VMEM((KG, IDXW), jnp.int32),
        pltpu.VMEM((CHUNK, DIM), jnp.float32),
        pltpu.SemaphoreType.DMA,
    ],
)(_body)


def kernel(indices, embedding_table):
    idx2d = indices.astype(jnp.int32).reshape(ROWS, IDXW)
    out = _emb(idx2d, embedding_table)
    return out.reshape(BATCH, N_FIELDS, DIM)


# speedup vs baseline: 1.5467x; 1.5467x over previous
import functools

import jax
import jax.numpy as jnp
from jax import lax
from jax.experimental import pallas as pl
from jax.experimental.pallas import tpu as pltpu
from jax.experimental.pallas import tpu_sc as plsc

BATCH = 16384
N_FIELDS = 26
DIM = 32
TOTAL = BATCH * N_FIELDS
IDXW = 128
ROWS = TOTAL // IDXW
NC = 2
NS = 16
NW = NC * NS
ROWS_PER_W = ROWS // NW
KG = 8
NCHUNK = ROWS_PER_W // KG
CHUNK = KG * IDXW


def _body(idx_hbm, table_hbm, out_hbm, idx_v, rows_v, sem):
    wid = lax.axis_index("s") * NC + lax.axis_index("c")
    row0 = wid * ROWS_PER_W

    def chunk(i, _):
        r = row0 + i * KG
        pltpu.sync_copy(idx_hbm.at[pl.ds(r, KG)], idx_v)
        copies = []
        for j in range(KG):
            copies.append(
                pltpu.async_copy(
                    table_hbm.at[idx_v.at[j]],
                    rows_v.at[pl.ds(j * IDXW, IDXW)],
                    sem,
                )
            )
        for c in copies:
            c.wait()
        pltpu.sync_copy(rows_v, out_hbm.at[pl.ds(r * IDXW, CHUNK)])
        return 0

    lax.fori_loop(0, NCHUNK, chunk, 0)


_emb = functools.partial(
    pl.kernel,
    out_type=jax.ShapeDtypeStruct((TOTAL, DIM), jnp.float32),
    mesh=plsc.VectorSubcoreMesh(core_axis_name="c", subcore_axis_name="s"),
    scratch_types=[
        pltpu.VMEM((KG, IDXW), jnp.int32),
        pltpu.VMEM((CHUNK, DIM), jnp.float32),
        pltpu.SemaphoreType.DMA,
    ],
    compiler_params=pltpu.CompilerParams(use_tc_tiling_on_sc=False),
)(_body)


def kernel(indices, embedding_table):
    idx2d = indices.astype(jnp.int32).reshape(ROWS, IDXW)
    out = _emb(idx2d, embedding_table)
    return out.reshape(BATCH, N_FIELDS, DIM)


# speedup vs baseline: 1.5754x; 1.0186x over previous
import functools

import jax
import jax.numpy as jnp
from jax import lax
from jax.experimental import pallas as pl
from jax.experimental.pallas import tpu as pltpu
from jax.experimental.pallas import tpu_sc as plsc

BATCH = 16384
N_FIELDS = 26
DIM = 32
TOTAL = BATCH * N_FIELDS
IDXW = 128
ROWS = TOTAL // IDXW
NC = 2
NS = 16
NW = NC * NS
ROWS_PER_W = ROWS // NW
KG = 4
NCHUNK = ROWS_PER_W // KG
CHUNK = KG * IDXW
NBUF = 4


def _body(idx_hbm, table_hbm, out_hbm, idx_v, rows_v, sem_g, sem_s):
    wid = lax.axis_index("s") * NC + lax.axis_index("c")
    row0 = wid * ROWS_PER_W

    pltpu.sync_copy(idx_hbm.at[pl.ds(row0, ROWS_PER_W)], idx_v)

    def fire_gathers(g, buf):
        for j in range(KG):
            pltpu.async_copy(
                table_hbm.at[idx_v.at[g * KG + j]],
                rows_v.at[buf].at[pl.ds(j * IDXW, IDXW)],
                sem_g,
            )

    def drain_gathers(buf):
        pltpu.make_async_copy(
            out_hbm.at[pl.ds(0, CHUNK)], rows_v.at[buf], sem_g
        ).wait()

    def wait_store(buf):
        pltpu.make_async_copy(
            rows_v.at[buf], out_hbm.at[pl.ds(0, CHUNK)], sem_s.at[buf]
        ).wait()

    fire_gathers(0, 0)

    def step(g, _):
        buf = lax.rem(g, NBUF)
        nbuf_i = lax.rem(g + 1, NBUF)

        @pl.when(jnp.logical_and(g + 1 < NCHUNK, g >= NBUF - 1))
        def _():
            wait_store(nbuf_i)

        @pl.when(g + 1 < NCHUNK)
        def _():
            fire_gathers(g + 1, nbuf_i)

        drain_gathers(buf)
        pltpu.async_copy(
            rows_v.at[buf],
            out_hbm.at[pl.ds((row0 + g * KG) * IDXW, CHUNK)],
            sem_s.at[buf],
        )
        return 0

    lax.fori_loop(0, NCHUNK, step, 0)

    for b in range(NBUF):
        wait_store(b)


_emb = functools.partial(
    pl.kernel,
    out_type=jax.ShapeDtypeStruct((TOTAL, DIM), jnp.float32),
    mesh=plsc.VectorSubcoreMesh(core_axis_name="c", subcore_axis_name="s"),
    scratch_types=[
        pltpu.VMEM((ROWS_PER_W, IDXW), jnp.int32),
        pltpu.VMEM((NBUF, CHUNK, DIM), jnp.float32),
        pltpu.SemaphoreType.DMA,
        pltpu.SemaphoreType.DMA((NBUF,)),
    ],
    compiler_params=pltpu.CompilerParams(use_tc_tiling_on_sc=False),
)(_body)


def kernel(indices, embedding_table):
    idx2d = indices.astype(jnp.int32).reshape(ROWS, IDXW)
    out = _emb(idx2d, embedding_table)
    return out.reshape(BATCH, N_FIELDS, DIM)
